# Initial kernel scaffold; baseline (speedup 1.0000x reference)
import jax
import jax.numpy as jnp
from jax.experimental import pallas as pl


def kernel(query, key, value, W_ref, b_ref, W_off, b_off, W_v, b_v, W_out, b_out):
    raise NotImplementedError("write your pallas kernel here")


# speedup vs baseline: 5.3436x; 5.3436x over previous
import functools

import jax
import jax.numpy as jnp
from jax import lax
from jax.experimental import pallas as pl
from jax.experimental.pallas import tpu as pltpu

H = 16
P = 4
C = 2048
Cph = C // H


def _mm_body(a_ref, b_ref, o_ref):
    o_ref[...] = lax.dot_general(
        a_ref[...], b_ref[...],
        (((1,), (1,)), ((), ())),
        preferred_element_type=jnp.float32,
    )


def _matmul_wt(a, w, bm=512, bn=512):
    M, K = a.shape
    N = w.shape[0]
    a16 = a.astype(jnp.bfloat16)
    w16 = w.astype(jnp.bfloat16)
    return pl.pallas_call(
        _mm_body,
        grid=(M // bm, N // bn),
        in_specs=[
            pl.BlockSpec((bm, K), lambda i, j: (i, 0)),
            pl.BlockSpec((bn, K), lambda i, j: (j, 0)),
        ],
        out_specs=pl.BlockSpec((bm, bn), lambda i, j: (i, j)),
        out_shape=jax.ShapeDtypeStruct((M, N), jnp.float32),
    )(a16, w16)


def kernel(query, key, value, W_ref, b_ref, W_off, b_off, W_v, b_v, W_out, b_out):
    N = query.shape[0]
    M = key.shape[0]

    ref = jax.nn.sigmoid(query @ W_ref.T + b_ref).reshape(N, H, P, 2)
    off = (query @ W_off.T + b_off).reshape(N, H, P, 2)
    loc = ref + off
    x = loc[..., 0] * (M - 1)
    y = loc[..., 1] * (M - 1)
    x0f = jnp.floor(x).astype(jnp.int32)
    y0f = jnp.floor(y).astype(jnp.int32)
    x0 = jnp.clip(x0f, 0, M - 1)
    x1 = jnp.clip(x0f + 1, 0, M - 1)
    y0 = jnp.clip(y0f, 0, M - 1)
    wx = x - x0.astype(jnp.float32)
    wy = y - y0.astype(jnp.float32)
    c00 = (1 - wx) * (1 - wy)
    c10 = wx * (1 - wy)
    c01 = (1 - wx) * wy
    c11 = wx * wy

    h_ar = jnp.arange(H, dtype=jnp.int32)[None, :, None]
    g0 = x0 * H + h_ar
    g1 = x1 * H + h_ar

    vproj = _matmul_wt(value, W_v) + b_v

    fV = vproj.reshape(M * H, Cph)
    fK = key.reshape(M * H, Cph)
    V0 = fV[g0.reshape(-1)].reshape(N, H, P, Cph)
    V1 = fV[g1.reshape(-1)].reshape(N, H, P, Cph)
    K0 = fK[g0.reshape(-1)].reshape(N, H, P, Cph)
    K1 = fK[g1.reshape(-1)].reshape(N, H, P, Cph)

    def comb(a0, a1):
        t = c00[..., None] * a0
        t = t + c10[..., None] * a1
        t = t + c01[..., None] * a0
        t = t + c11[..., None] * a1
        return t

    ksamp = comb(K0, K1)
    sampled = comb(V0, V1)
    q_h = query.reshape(N, H, Cph)
    logits = jnp.einsum('nhc,nhpc->nhp', q_h, ksamp) / jnp.sqrt(jnp.float32(Cph))
    attn = jax.nn.softmax(logits, axis=-1)
    out = jnp.einsum('nhp,nhpc->nhc', attn, sampled).reshape(N, C)

    return _matmul_wt(out, W_out) + b_out


# speedup vs baseline: 14.4547x; 2.7050x over previous
import functools

import jax
import jax.numpy as jnp
from jax import lax
from jax.experimental import pallas as pl
from jax.experimental.pallas import tpu as pltpu
from jax.experimental.pallas import tpu_sc as plsc

H = 16
P = 4
C = 2048
Cph = C // H
NV = Cph // 16
TW = 2 * Cph


def _mm_body(a_ref, b_ref, o_ref):
    o_ref[...] = lax.dot_general(
        a_ref[...], b_ref[...],
        (((1,), (1,)), ((), ())),
        preferred_element_type=jnp.float32,
    )


def _matmul_wt(a, w, bm=512, bn=512):
    M, K = a.shape
    N = w.shape[0]
    return pl.pallas_call(
        _mm_body,
        grid=(M // bm, N // bn),
        in_specs=[
            pl.BlockSpec((bm, K), lambda i, j: (i, 0)),
            pl.BlockSpec((bn, K), lambda i, j: (j, 0)),
        ],
        out_specs=pl.BlockSpec((bm, bn), lambda i, j: (i, j)),
        out_shape=jax.ShapeDtypeStruct((M, N), jnp.float32),
    )(a.astype(jnp.bfloat16), w.astype(jnp.bfloat16))


def _table_body(v_ref, wv_ref, k_ref, bv_ref, o_ref):
    mm = lax.dot_general(
        v_ref[...], wv_ref[...],
        (((1,), (1,)), ((), ())),
        preferred_element_type=jnp.float32,
    )
    o_ref[0, :, :Cph] = mm + bv_ref[0]
    o_ref[0, :, Cph:] = k_ref[...]


def _build_table(value, W_v, key, b_v, bm=512):
    M, K = value.shape
    return pl.pallas_call(
        _table_body,
        grid=(M // bm, H),
        in_specs=[
            pl.BlockSpec((bm, K), lambda i, j: (i, 0)),
            pl.BlockSpec((Cph, K), lambda i, j: (j, 0)),
            pl.BlockSpec((bm, Cph), lambda i, j: (i, j)),
            pl.BlockSpec((1, 1, Cph), lambda i, j: (j, 0, 0)),
        ],
        out_specs=pl.BlockSpec((1, bm, TW), lambda i, j: (j, i, 0)),
        out_shape=jax.ShapeDtypeStruct((H, M, TW), jnp.float32),
    )(value.astype(jnp.bfloat16), W_v.astype(jnp.bfloat16), key,
      b_v.reshape(H, 1, Cph))


_INV_SQRT_CPH = jnp.float32(0.08838834764831845)


def _rtne_bf16(x):
    b = lax.bitcast_convert_type(x, jnp.uint32)
    r = (b + jnp.uint32(0x7FFF) + ((b >> jnp.uint32(16)) & jnp.uint32(1))) \
        & jnp.uint32(0xFFFF0000)
    return lax.bitcast_convert_type(r, jnp.float32)


_GDN = lax.GatherDimensionNumbers(
    offset_dims=(), collapsed_slice_dims=(0,), start_index_map=(0,))


def _lane_gather(v, idx):
    return lax.gather(v, idx[:, None], _GDN, (1,),
                      mode=lax.GatherScatterMode.PROMISE_IN_BOUNDS)


def _bfly(v, op):
    iota = lax.iota(jnp.int32, 16)
    for m in (1, 2, 4, 8):
        v = op(v, _lane_gather(v, jnp.bitwise_xor(iota, m)))
    return v


def _sc_attend(T2d, q2d, gidx, cexp, N):
    info = plsc.get_sparse_core_info()
    NC, NS = info.num_cores, info.num_subcores
    NW = NC * NS
    QW = N // NW
    mesh = plsc.VectorSubcoreMesh(core_axis_name="c", subcore_axis_name="s")

    @functools.partial(
        pl.kernel, mesh=mesh,
        out_type=jax.ShapeDtypeStruct((N * H, Cph), jnp.float32),
        scratch_types=[
            pltpu.VMEM((Cph,), jnp.int32),
            pltpu.VMEM((H, Cph), jnp.float32),
            pltpu.VMEM((H * P * 4 * 16,), jnp.float32),
            pltpu.VMEM((Cph, TW), jnp.float32),
            pltpu.VMEM((H, Cph), jnp.float32),
            pltpu.VMEM((16,), jnp.float32),
            pltpu.SemaphoreType.DMA,
        ],
    )
    def k(T_hbm, q_hbm, gidx_hbm, cexp_hbm, zeros_hbm, out_hbm,
          idx_v, q_v, c_v, rows_v, out_v, z_v, sem):
        wid = lax.axis_index("c") * NS + lax.axis_index("s")
        base = wid * QW
        pltpu.sync_copy(zeros_hbm, z_v)

        def per_query(qi, _):
            n = base + qi
            pltpu.sync_copy(gidx_hbm.at[n], idx_v)
            pltpu.sync_copy(q_hbm.at[pl.ds(n * H, H)], q_v)
            pltpu.sync_copy(cexp_hbm.at[n], c_v)
            pltpu.async_copy(T_hbm.at[idx_v], rows_v, sem).wait()

            def per_head(h, _):
                rbase = h * (P * 2)
                cbase = h * (P * 4 * 16)
                iota = lax.iota(jnp.int32, 16)
                cs = []
                for p in range(P):
                    cs.append(tuple(
                        c_v[pl.ds(cbase + p * 64 + t * 16, 16)]
                        for t in range(4)))
                lvec = jnp.full((16,), -1e30, jnp.float32)
                zv = z_v[...]
                qb = [_rtne_bf16(q_v[h, pl.ds(i * 16, 16)]) for i in range(NV)]
                for p in range(P):
                    c00, c10, c01, c11 = cs[p]
                    acc = jnp.zeros((16,), jnp.float32)
                    for i in range(NV):
                        k0 = rows_v[rbase + 2 * p, pl.ds(Cph + i * 16, 16)]
                        k1 = rows_v[rbase + 2 * p + 1, pl.ds(Cph + i * 16, 16)]
                        ks = (c00 * k0 + zv) + (c10 * k1 + zv)
                        ks = ks + (c01 * k0 + zv)
                        ks = ks + (c11 * k1 + zv)
                        acc = acc + qb[i] * _rtne_bf16(ks)
                    tot = _bfly(acc, jnp.add) * _INV_SQRT_CPH
                    lvec = jnp.where(iota == p, tot, lvec)
                m = _bfly(lvec, jnp.maximum)
                e = jnp.exp(lvec - m)
                e = jnp.where(iota < P, e, jnp.zeros((16,), jnp.float32))
                attn = _rtne_bf16(e / _bfly(e, jnp.add))
                avs = [_lane_gather(attn, jnp.full((16,), p, jnp.int32))
                       for p in range(P)]
                for i in range(NV):
                    acc = jnp.zeros((16,), jnp.float32)
                    for p in range(P):
                        c00, c10, c01, c11 = cs[p]
                        v0 = rows_v[rbase + 2 * p, pl.ds(i * 16, 16)]
                        v1 = rows_v[rbase + 2 * p + 1, pl.ds(i * 16, 16)]
                        vs = (c00 * v0 + zv) + (c10 * v1 + zv)
                        vs = vs + (c01 * v0 + zv)
                        vs = vs + (c11 * v1 + zv)
                        acc = acc + avs[p] * _rtne_bf16(vs)
                    out_v[h, pl.ds(i * 16, 16)] = acc
                return 0

            lax.fori_loop(0, H, per_head, 0)
            pltpu.sync_copy(out_v, out_hbm.at[pl.ds(n * H, H)])
            return 0

        lax.fori_loop(0, QW, per_query, 0)

    return k(T2d, q2d, gidx, cexp, jnp.zeros((16,), jnp.float32))


def kernel(query, key, value, W_ref, b_ref, W_off, b_off, W_v, b_v, W_out, b_out):
    N = query.shape[0]
    M = key.shape[0]

    ref = jax.nn.sigmoid(query @ W_ref.T + b_ref).reshape(N, H, P, 2)
    off = (query @ W_off.T + b_off).reshape(N, H, P, 2)
    loc = ref + off
    x = loc[..., 0] * (M - 1)
    y = loc[..., 1] * (M - 1)
    x0f = jnp.floor(x).astype(jnp.int32)
    y0f = jnp.floor(y).astype(jnp.int32)
    x0 = jnp.clip(x0f, 0, M - 1)
    x1 = jnp.clip(x0f + 1, 0, M - 1)
    y0 = jnp.clip(y0f, 0, M - 1)
    wx = x - x0.astype(jnp.float32)
    wy = y - y0.astype(jnp.float32)
    c00 = (1 - wx) * (1 - wy)
    c10 = wx * (1 - wy)
    c01 = (1 - wx) * wy
    c11 = wx * wy

    h_ar = jnp.arange(H, dtype=jnp.int32)[None, :, None]
    g0 = x0 * H + h_ar
    g1 = x1 * H + h_ar
    gidx = jnp.stack([g0, g1], axis=-1).reshape(N, H * P * 2).astype(jnp.int32)
    cexp = jnp.broadcast_to(
        jnp.stack([c00, c10, c01, c11], axis=-1)[..., None],
        (N, H, P, 4, 16)).reshape(N, H * P * 4 * 16)

    vproj = _matmul_wt(value, W_v) + b_v
    T = jnp.concatenate(
        [vproj.reshape(M, H, Cph), key.reshape(M, H, Cph)], axis=2
    ).reshape(M * H, TW)

    out2d = _sc_attend(T, query.reshape(N * H, Cph), gidx, cexp, N)

    return _matmul_wt(out2d.reshape(N, C), W_out) + b_out


# speedup vs baseline: 15.5591x; 1.0764x over previous
import functools

import jax
import jax.numpy as jnp
from jax import lax
from jax.experimental import pallas as pl
from jax.experimental.pallas import tpu as pltpu
from jax.experimental.pallas import tpu_sc as plsc

H = 16
P = 4
C = 2048
Cph = C // H
NV = Cph // 16
TW = 2 * Cph


def _mm_body(a_ref, b_ref, o_ref):
    o_ref[...] = lax.dot_general(
        a_ref[...], b_ref[...],
        (((1,), (1,)), ((), ())),
        preferred_element_type=jnp.float32,
    )


def _matmul_wt(a, w, bm=512, bn=512):
    M, K = a.shape
    N = w.shape[0]
    return pl.pallas_call(
        _mm_body,
        grid=(M // bm, N // bn),
        in_specs=[
            pl.BlockSpec((bm, K), lambda i, j: (i, 0)),
            pl.BlockSpec((bn, K), lambda i, j: (j, 0)),
        ],
        out_specs=pl.BlockSpec((bm, bn), lambda i, j: (i, j)),
        out_shape=jax.ShapeDtypeStruct((M, N), jnp.float32),
    )(a.astype(jnp.bfloat16), w.astype(jnp.bfloat16))


def _table_body(v_ref, wv_ref, k_ref, bv_ref, o_ref):
    mm = lax.dot_general(
        v_ref[...], wv_ref[...],
        (((1,), (1,)), ((), ())),
        preferred_element_type=jnp.float32,
    )
    o_ref[0, :, :Cph] = mm + bv_ref[0]
    o_ref[0, :, Cph:] = k_ref[...]


def _build_table(value, W_v, key, b_v, bm=512):
    M, K = value.shape
    return pl.pallas_call(
        _table_body,
        grid=(M // bm, H),
        in_specs=[
            pl.BlockSpec((bm, K), lambda i, j: (i, 0)),
            pl.BlockSpec((Cph, K), lambda i, j: (j, 0)),
            pl.BlockSpec((bm, Cph), lambda i, j: (i, j)),
            pl.BlockSpec((1, 1, Cph), lambda i, j: (j, 0, 0)),
        ],
        out_specs=pl.BlockSpec((1, bm, TW), lambda i, j: (j, i, 0)),
        out_shape=jax.ShapeDtypeStruct((H, M, TW), jnp.float32),
    )(value.astype(jnp.bfloat16), W_v.astype(jnp.bfloat16), key,
      b_v.reshape(H, 1, Cph))


_INV_SQRT_CPH = jnp.float32(0.08838834764831845)


def _rtne_bf16(x):
    b = lax.bitcast_convert_type(x, jnp.uint32)
    r = (b + jnp.uint32(0x7FFF) + ((b >> jnp.uint32(16)) & jnp.uint32(1))) \
        & jnp.uint32(0xFFFF0000)
    return lax.bitcast_convert_type(r, jnp.float32)


_GDN = lax.GatherDimensionNumbers(
    offset_dims=(), collapsed_slice_dims=(0,), start_index_map=(0,))


def _lane_gather(v, idx):
    return lax.gather(v, idx[:, None], _GDN, (1,),
                      mode=lax.GatherScatterMode.PROMISE_IN_BOUNDS)


def _bfly(v, op):
    iota = lax.iota(jnp.int32, 16)
    for m in (1, 2, 4, 8):
        v = op(v, _lane_gather(v, jnp.bitwise_xor(iota, m)))
    return v


def _sc_attend(T2d, q2d, gidx, cexp, N):
    info = plsc.get_sparse_core_info()
    NC, NS = info.num_cores, info.num_subcores
    NW = NC * NS
    QW = N // NW
    mesh = plsc.VectorSubcoreMesh(core_axis_name="c", subcore_axis_name="s")

    @functools.partial(
        pl.kernel, mesh=mesh,
        out_type=jax.ShapeDtypeStruct((N * H, Cph), jnp.float32),
        scratch_types=[
            pltpu.VMEM((QW, Cph), jnp.int32),
            pltpu.VMEM((QW, H * P * 4), jnp.float32),
            pltpu.VMEM((2, H, Cph), jnp.float32),
            pltpu.VMEM((2, Cph, TW), jnp.float32),
            pltpu.VMEM((2, H, Cph), jnp.float32),
            pltpu.VMEM((16,), jnp.float32),
            pltpu.SemaphoreType.DMA,
            pltpu.SemaphoreType.DMA,
            pltpu.SemaphoreType.DMA,
            pltpu.SemaphoreType.DMA,
            pltpu.SemaphoreType.DMA,
            pltpu.SemaphoreType.DMA,
        ],
    )
    def k(T_hbm, q_hbm, gidx_hbm, cexp_hbm, zeros_hbm, out_hbm,
          idx_v, c_v, q_v, rows_v, out_v, z_v,
          sg0, sg1, sq0, sq1, sw0, sw1):
        wid = lax.axis_index("c") * NS + lax.axis_index("s")
        base = wid * QW
        pltpu.sync_copy(zeros_hbm, z_v)
        pltpu.sync_copy(gidx_hbm.at[pl.ds(base, QW)], idx_v)
        pltpu.sync_copy(cexp_hbm.at[pl.ds(base, QW)], c_v)

        def issue(qi, b, sg, sq):
            pltpu.async_copy(T_hbm.at[idx_v.at[qi]], rows_v.at[b], sg)
            pltpu.async_copy(q_hbm.at[pl.ds((base + qi) * H, H)],
                             q_v.at[b], sq)

        def compute(qi, b):
            def per_head(h, _):
                rbase = h * (P * 2)
                iota = lax.iota(jnp.int32, 16)
                cv = c_v[qi, pl.ds(h * 16, 16)]
                cs = [tuple(_lane_gather(cv, jnp.full((16,), p * 4 + t,
                                                      jnp.int32))
                            for t in range(4)) for p in range(P)]
                lvec = jnp.full((16,), -1e30, jnp.float32)
                zv = z_v[...]
                qb = [_rtne_bf16(q_v[b, h, pl.ds(i * 16, 16)])
                      for i in range(NV)]
                for p in range(P):
                    c00, c10, c01, c11 = cs[p]
                    acc = jnp.zeros((16,), jnp.float32)
                    for i in range(NV):
                        k0 = rows_v[b, rbase + 2 * p, pl.ds(Cph + i * 16, 16)]
                        k1 = rows_v[b, rbase + 2 * p + 1,
                                    pl.ds(Cph + i * 16, 16)]
                        ks = (c00 * k0 + zv) + (c10 * k1 + zv)
                        ks = ks + (c01 * k0 + zv)
                        ks = ks + (c11 * k1 + zv)
                        acc = acc + qb[i] * _rtne_bf16(ks)
                    tot = _bfly(acc, jnp.add) * _INV_SQRT_CPH
                    lvec = jnp.where(iota == p, tot, lvec)
                m = _bfly(lvec, jnp.maximum)
                e = jnp.exp(lvec - m)
                e = jnp.where(iota < P, e, jnp.zeros((16,), jnp.float32))
                attn = _rtne_bf16(e / _bfly(e, jnp.add))
                avs = [_lane_gather(attn, jnp.full((16,), p, jnp.int32))
                       for p in range(P)]
                for i in range(NV):
                    acc = jnp.zeros((16,), jnp.float32)
                    for p in range(P):
                        c00, c10, c01, c11 = cs[p]
                        v0 = rows_v[b, rbase + 2 * p, pl.ds(i * 16, 16)]
                        v1 = rows_v[b, rbase + 2 * p + 1, pl.ds(i * 16, 16)]
                        vs = (c00 * v0 + zv) + (c10 * v1 + zv)
                        vs = vs + (c01 * v0 + zv)
                        vs = vs + (c11 * v1 + zv)
                        acc = acc + avs[p] * _rtne_bf16(vs)
                    out_v[b, h, pl.ds(i * 16, 16)] = acc
                return 0

            lax.fori_loop(0, H, per_head, 0)

        def wait_g(qi, b, sg, sq):
            pltpu.make_async_copy(T_hbm.at[idx_v.at[qi]], rows_v.at[b],
                                  sg).wait()
            pltpu.make_async_copy(q_hbm.at[pl.ds((base + qi) * H, H)],
                                  q_v.at[b], sq).wait()

        def write(qi, b, sw):
            pltpu.async_copy(out_v.at[b],
                             out_hbm.at[pl.ds((base + qi) * H, H)], sw)

        def wait_w(qi, b, sw):
            pltpu.make_async_copy(out_v.at[b],
                                  out_hbm.at[pl.ds((base + qi) * H, H)],
                                  sw).wait()

        issue(0, 0, sg0, sq0)

        def pair(qi2, _):
            q0 = qi2 * 2
            q1 = q0 + 1
            issue(q1, 1, sg1, sq1)
            wait_g(q0, 0, sg0, sq0)

            @pl.when(qi2 >= 1)
            def _():
                wait_w(q0 - 2, 0, sw0)
            compute(q0, 0)
            write(q0, 0, sw0)

            @pl.when(q1 + 1 < QW)
            def _():
                issue(q1 + 1, 0, sg0, sq0)
            wait_g(q1, 1, sg1, sq1)

            @pl.when(qi2 >= 1)
            def _():
                wait_w(q1 - 2, 1, sw1)
            compute(q1, 1)
            write(q1, 1, sw1)
            return 0

        lax.fori_loop(0, QW // 2, pair, 0)
        wait_w(QW - 2, 0, sw0)
        wait_w(QW - 1, 1, sw1)

    return k(T2d, q2d, gidx, cexp, jnp.zeros((16,), jnp.float32))


def kernel(query, key, value, W_ref, b_ref, W_off, b_off, W_v, b_v, W_out, b_out):
    N = query.shape[0]
    M = key.shape[0]

    ref = jax.nn.sigmoid(query @ W_ref.T + b_ref).reshape(N, H, P, 2)
    off = (query @ W_off.T + b_off).reshape(N, H, P, 2)
    loc = ref + off
    x = loc[..., 0] * (M - 1)
    y = loc[..., 1] * (M - 1)
    x0f = jnp.floor(x).astype(jnp.int32)
    y0f = jnp.floor(y).astype(jnp.int32)
    x0 = jnp.clip(x0f, 0, M - 1)
    x1 = jnp.clip(x0f + 1, 0, M - 1)
    y0 = jnp.clip(y0f, 0, M - 1)
    wx = x - x0.astype(jnp.float32)
    wy = y - y0.astype(jnp.float32)
    c00 = (1 - wx) * (1 - wy)
    c10 = wx * (1 - wy)
    c01 = (1 - wx) * wy
    c11 = wx * wy

    h_ar = jnp.arange(H, dtype=jnp.int32)[None, :, None]
    g0 = x0 * H + h_ar
    g1 = x1 * H + h_ar
    gidx = jnp.stack([g0, g1], axis=-1).reshape(N, H * P * 2).astype(jnp.int32)
    cexp = jnp.stack([c00, c10, c01, c11], axis=-1).reshape(N, H * P * 4)

    vproj = _matmul_wt(value, W_v) + b_v
    T = jnp.concatenate(
        [vproj.reshape(M, H, Cph), key.reshape(M, H, Cph)], axis=2
    ).reshape(M * H, TW)

    out2d = _sc_attend(T, query.reshape(N * H, Cph), gidx, cexp, N)

    return _matmul_wt(out2d.reshape(N, C), W_out) + b_out


# speedup vs baseline: 15.9390x; 1.0244x over previous
import functools

import jax
import jax.numpy as jnp
from jax import lax
from jax.experimental import pallas as pl
from jax.experimental.pallas import tpu as pltpu
from jax.experimental.pallas import tpu_sc as plsc

H = 16
P = 4
C = 2048
Cph = C // H
NV = Cph // 16
TW = 2 * Cph


def _mm_body(a_ref, b_ref, o_ref):
    o_ref[...] = lax.dot_general(
        a_ref[...], b_ref[...],
        (((1,), (1,)), ((), ())),
        preferred_element_type=jnp.float32,
    )


def _matmul_wt(a, w, bm=512, bn=512):
    M, K = a.shape
    N = w.shape[0]
    return pl.pallas_call(
        _mm_body,
        grid=(M // bm, N // bn),
        in_specs=[
            pl.BlockSpec((bm, K), lambda i, j: (i, 0)),
            pl.BlockSpec((bn, K), lambda i, j: (j, 0)),
        ],
        out_specs=pl.BlockSpec((bm, bn), lambda i, j: (i, j)),
        out_shape=jax.ShapeDtypeStruct((M, N), jnp.float32),
    )(a.astype(jnp.bfloat16), w.astype(jnp.bfloat16))


def _table_body(v_ref, wv_ref, k_ref, bv_ref, o_ref):
    mm = lax.dot_general(
        v_ref[...], wv_ref[...],
        (((1,), (1,)), ((), ())),
        preferred_element_type=jnp.float32,
    )
    o_ref[0, :, :Cph] = mm + bv_ref[0]
    o_ref[0, :, Cph:] = k_ref[...]


def _build_table(value, W_v, key, b_v, bm=512):
    M, K = value.shape
    return pl.pallas_call(
        _table_body,
        grid=(M // bm, H),
        in_specs=[
            pl.BlockSpec((bm, K), lambda i, j: (i, 0)),
            pl.BlockSpec((Cph, K), lambda i, j: (j, 0)),
            pl.BlockSpec((bm, Cph), lambda i, j: (i, j)),
            pl.BlockSpec((1, 1, Cph), lambda i, j: (j, 0, 0)),
        ],
        out_specs=pl.BlockSpec((1, bm, TW), lambda i, j: (j, i, 0)),
        out_shape=jax.ShapeDtypeStruct((H, M, TW), jnp.float32),
    )(value.astype(jnp.bfloat16), W_v.astype(jnp.bfloat16), key,
      b_v.reshape(H, 1, Cph))


_INV_SQRT_CPH = jnp.float32(0.08838834764831845)


def _rtne_bf16(x):
    b = lax.bitcast_convert_type(x, jnp.uint32)
    r = (b + jnp.uint32(0x7FFF) + ((b >> jnp.uint32(16)) & jnp.uint32(1))) \
        & jnp.uint32(0xFFFF0000)
    return lax.bitcast_convert_type(r, jnp.float32)


_GDN = lax.GatherDimensionNumbers(
    offset_dims=(), collapsed_slice_dims=(0,), start_index_map=(0,))


def _lane_gather(v, idx):
    return lax.gather(v, idx[:, None], _GDN, (1,),
                      mode=lax.GatherScatterMode.PROMISE_IN_BOUNDS)


def _bfly(v, op):
    iota = lax.iota(jnp.int32, 16)
    for m in (1, 2, 4, 8):
        v = op(v, _lane_gather(v, jnp.bitwise_xor(iota, m)))
    return v


def _sc_attend(T2d, q2d, gidx, cexp, N):
    info = plsc.get_sparse_core_info()
    NC, NS = info.num_cores, info.num_subcores
    NW = NC * NS
    QW = N // NW
    mesh = plsc.VectorSubcoreMesh(core_axis_name="c", subcore_axis_name="s")

    @functools.partial(
        pl.kernel, mesh=mesh,
        out_type=jax.ShapeDtypeStruct((N * H, Cph), jnp.float32),
        scratch_types=[
            pltpu.VMEM((QW, Cph), jnp.int32),
            pltpu.VMEM((QW, H * P * 4), jnp.float32),
            pltpu.VMEM((2, H, Cph), jnp.float32),
            pltpu.VMEM((2, Cph, TW), jnp.float32),
            pltpu.VMEM((2, H, Cph), jnp.float32),
            pltpu.VMEM((16,), jnp.float32),
            pltpu.SemaphoreType.DMA,
            pltpu.SemaphoreType.DMA,
            pltpu.SemaphoreType.DMA,
            pltpu.SemaphoreType.DMA,
            pltpu.SemaphoreType.DMA,
            pltpu.SemaphoreType.DMA,
        ],
    )
    def k(T_hbm, q_hbm, gidx_hbm, cexp_hbm, zeros_hbm, out_hbm,
          idx_v, c_v, q_v, rows_v, out_v, z_v,
          sg0, sg1, sq0, sq1, sw0, sw1):
        wid = lax.axis_index("c") * NS + lax.axis_index("s")
        base = wid * QW
        pltpu.sync_copy(zeros_hbm, z_v)
        pltpu.sync_copy(gidx_hbm.at[pl.ds(base, QW)], idx_v)
        pltpu.sync_copy(cexp_hbm.at[pl.ds(base, QW)], c_v)

        def issue(qi, b, sg, sq):
            pltpu.async_copy(T_hbm.at[idx_v.at[qi]], rows_v.at[b], sg)
            pltpu.async_copy(q_hbm.at[pl.ds((base + qi) * H, H)],
                             q_v.at[b], sq)

        def compute(qi, b):
            def per_head(h, _):
                rbase = h * (P * 2)
                iota = lax.iota(jnp.int32, 16)
                cv = c_v[qi, pl.ds(h * 16, 16)]
                cs = [tuple(_lane_gather(cv, jnp.full((16,), p * 4 + t,
                                                      jnp.int32))
                            for t in range(4)) for p in range(P)]
                lvec = jnp.full((16,), -1e30, jnp.float32)
                zv = z_v[...]
                qb = [q_v[b, h, pl.ds(i * 16, 16)] for i in range(NV)]
                for p in range(P):
                    c00, c10, c01, c11 = cs[p]
                    acc = jnp.zeros((16,), jnp.float32)
                    for i in range(NV):
                        k0 = rows_v[b, rbase + 2 * p, pl.ds(Cph + i * 16, 16)]
                        k1 = rows_v[b, rbase + 2 * p + 1,
                                    pl.ds(Cph + i * 16, 16)]
                        ks = (c00 * k0 + zv) + (c10 * k1 + zv)
                        ks = ks + (c01 * k0 + zv)
                        ks = ks + (c11 * k1 + zv)
                        acc = acc + qb[i] * _rtne_bf16(ks)
                    tot = _bfly(acc, jnp.add) * _INV_SQRT_CPH
                    lvec = jnp.where(iota == p, tot, lvec)
                m = _bfly(lvec, jnp.maximum)
                e = jnp.exp(lvec - m)
                e = jnp.where(iota < P, e, jnp.zeros((16,), jnp.float32))
                attn = _rtne_bf16(e / _bfly(e, jnp.add))
                avs = [_lane_gather(attn, jnp.full((16,), p, jnp.int32))
                       for p in range(P)]
                for i in range(NV):
                    acc = jnp.zeros((16,), jnp.float32)
                    for p in range(P):
                        c00, c10, c01, c11 = cs[p]
                        v0 = rows_v[b, rbase + 2 * p, pl.ds(i * 16, 16)]
                        v1 = rows_v[b, rbase + 2 * p + 1, pl.ds(i * 16, 16)]
                        vs = (c00 * v0 + zv) + (c10 * v1 + zv)
                        vs = vs + (c01 * v0 + zv)
                        vs = vs + (c11 * v1 + zv)
                        acc = acc + avs[p] * _rtne_bf16(vs)
                    out_v[b, h, pl.ds(i * 16, 16)] = acc
                return 0

            lax.fori_loop(0, H, per_head, 0)

        def wait_g(qi, b, sg, sq):
            pltpu.make_async_copy(T_hbm.at[idx_v.at[qi]], rows_v.at[b],
                                  sg).wait()
            pltpu.make_async_copy(q_hbm.at[pl.ds((base + qi) * H, H)],
                                  q_v.at[b], sq).wait()

        def write(qi, b, sw):
            pltpu.async_copy(out_v.at[b],
                             out_hbm.at[pl.ds((base + qi) * H, H)], sw)

        def wait_w(qi, b, sw):
            pltpu.make_async_copy(out_v.at[b],
                                  out_hbm.at[pl.ds((base + qi) * H, H)],
                                  sw).wait()

        issue(0, 0, sg0, sq0)

        def pair(qi2, _):
            q0 = qi2 * 2
            q1 = q0 + 1
            issue(q1, 1, sg1, sq1)
            wait_g(q0, 0, sg0, sq0)

            @pl.when(qi2 >= 1)
            def _():
                wait_w(q0 - 2, 0, sw0)
            compute(q0, 0)
            write(q0, 0, sw0)

            @pl.when(q1 + 1 < QW)
            def _():
                issue(q1 + 1, 0, sg0, sq0)
            wait_g(q1, 1, sg1, sq1)

            @pl.when(qi2 >= 1)
            def _():
                wait_w(q1 - 2, 1, sw1)
            compute(q1, 1)
            write(q1, 1, sw1)
            return 0

        lax.fori_loop(0, QW // 2, pair, 0)
        wait_w(QW - 2, 0, sw0)
        wait_w(QW - 1, 1, sw1)

    return k(T2d, q2d, gidx, cexp, jnp.zeros((16,), jnp.float32))


def kernel(query, key, value, W_ref, b_ref, W_off, b_off, W_v, b_v, W_out, b_out):
    N = query.shape[0]
    M = key.shape[0]

    ref = jax.nn.sigmoid(query @ W_ref.T + b_ref).reshape(N, H, P, 2)
    off = (query @ W_off.T + b_off).reshape(N, H, P, 2)
    loc = ref + off
    x = loc[..., 0] * (M - 1)
    y = loc[..., 1] * (M - 1)
    x0f = jnp.floor(x).astype(jnp.int32)
    y0f = jnp.floor(y).astype(jnp.int32)
    x0 = jnp.clip(x0f, 0, M - 1)
    x1 = jnp.clip(x0f + 1, 0, M - 1)
    y0 = jnp.clip(y0f, 0, M - 1)
    wx = x - x0.astype(jnp.float32)
    wy = y - y0.astype(jnp.float32)
    c00 = (1 - wx) * (1 - wy)
    c10 = wx * (1 - wy)
    c01 = (1 - wx) * wy
    c11 = wx * wy

    h_ar = jnp.arange(H, dtype=jnp.int32)[None, :, None]
    g0 = x0 * H + h_ar
    g1 = x1 * H + h_ar
    gidx = jnp.stack([g0, g1], axis=-1).reshape(N, H * P * 2).astype(jnp.int32)
    cexp = jnp.stack([c00, c10, c01, c11], axis=-1).reshape(N, H * P * 4)

    vproj = _matmul_wt(value, W_v) + b_v
    T = jnp.concatenate(
        [vproj.reshape(M, H, Cph), key.reshape(M, H, Cph)], axis=2
    ).reshape(M * H, TW)

    q_r = query.astype(jnp.bfloat16).astype(jnp.float32)
    out2d = _sc_attend(T, q_r.reshape(N * H, Cph), gidx, cexp, N)

    return _matmul_wt(out2d.reshape(N, C), W_out) + b_out


# speedup vs baseline: 16.0860x; 1.0092x over previous
import functools

import jax
import jax.numpy as jnp
from jax import lax
from jax.experimental import pallas as pl
from jax.experimental.pallas import tpu as pltpu
from jax.experimental.pallas import tpu_sc as plsc

H = 16
P = 4
C = 2048
Cph = C // H
NV = Cph // 16
TW = 2 * Cph


def _mm_body(a_ref, b_ref, o_ref):
    o_ref[...] = lax.dot_general(
        a_ref[...], b_ref[...],
        (((1,), (1,)), ((), ())),
        preferred_element_type=jnp.float32,
    )


def _matmul_wt(a, w, bm=512, bn=512):
    M, K = a.shape
    N = w.shape[0]
    return pl.pallas_call(
        _mm_body,
        grid=(M // bm, N // bn),
        in_specs=[
            pl.BlockSpec((bm, K), lambda i, j: (i, 0)),
            pl.BlockSpec((bn, K), lambda i, j: (j, 0)),
        ],
        out_specs=pl.BlockSpec((bm, bn), lambda i, j: (i, j)),
        out_shape=jax.ShapeDtypeStruct((M, N), jnp.float32),
    )(a.astype(jnp.bfloat16), w.astype(jnp.bfloat16))


def _table_body(v_ref, wv_ref, k_ref, bv_ref, o_ref):
    mm = lax.dot_general(
        v_ref[...], wv_ref[...],
        (((1,), (1,)), ((), ())),
        preferred_element_type=jnp.float32,
    )
    o_ref[0, :, :Cph] = mm + bv_ref[0]
    o_ref[0, :, Cph:] = k_ref[...]


def _build_table(value, W_v, key, b_v, bm=512):
    M, K = value.shape
    return pl.pallas_call(
        _table_body,
        grid=(M // bm, H),
        in_specs=[
            pl.BlockSpec((bm, K), lambda i, j: (i, 0)),
            pl.BlockSpec((Cph, K), lambda i, j: (j, 0)),
            pl.BlockSpec((bm, Cph), lambda i, j: (i, j)),
            pl.BlockSpec((1, 1, Cph), lambda i, j: (j, 0, 0)),
        ],
        out_specs=pl.BlockSpec((1, bm, TW), lambda i, j: (j, i, 0)),
        out_shape=jax.ShapeDtypeStruct((H, M, TW), jnp.float32),
    )(value.astype(jnp.bfloat16), W_v.astype(jnp.bfloat16), key,
      b_v.reshape(H, 1, Cph))


_INV_SQRT_CPH = jnp.float32(0.08838834764831845)


def _rtne_bf16(x):
    b = lax.bitcast_convert_type(x, jnp.uint32)
    r = (b + jnp.uint32(0x7FFF) + ((b >> jnp.uint32(16)) & jnp.uint32(1))) \
        & jnp.uint32(0xFFFF0000)
    return lax.bitcast_convert_type(r, jnp.float32)


_GDN = lax.GatherDimensionNumbers(
    offset_dims=(), collapsed_slice_dims=(0,), start_index_map=(0,))


def _lane_gather(v, idx):
    return lax.gather(v, idx[:, None], _GDN, (1,),
                      mode=lax.GatherScatterMode.PROMISE_IN_BOUNDS)


def _bfly(v, op):
    iota = lax.iota(jnp.int32, 16)
    for m in (1, 2, 4, 8):
        v = op(v, _lane_gather(v, jnp.bitwise_xor(iota, m)))
    return v


def _sc_attend(T2d, q2d, gidx, cexp, N):
    info = plsc.get_sparse_core_info()
    NC, NS = info.num_cores, info.num_subcores
    NW = NC * NS
    QW = N // NW
    mesh = plsc.VectorSubcoreMesh(core_axis_name="c", subcore_axis_name="s")

    @functools.partial(
        pl.kernel, mesh=mesh,
        out_type=jax.ShapeDtypeStruct((N * H, Cph), jnp.float32),
        scratch_types=[
            pltpu.VMEM((QW, Cph), jnp.int32),
            pltpu.VMEM((QW, H * P * 4), jnp.float32),
            pltpu.VMEM((2, H, Cph), jnp.float32),
            pltpu.VMEM((2, Cph, TW), jnp.float32),
            pltpu.VMEM((2, H, Cph), jnp.float32),
            pltpu.VMEM((16,), jnp.float32),
            pltpu.SemaphoreType.DMA,
            pltpu.SemaphoreType.DMA,
            pltpu.SemaphoreType.DMA,
            pltpu.SemaphoreType.DMA,
            pltpu.SemaphoreType.DMA,
            pltpu.SemaphoreType.DMA,
        ],
    )
    def k(T_hbm, q_hbm, gidx_hbm, cexp_hbm, zeros_hbm, out_hbm,
          idx_v, c_v, q_v, rows_v, out_v, z_v,
          sg0, sg1, sq0, sq1, sw0, sw1):
        wid = lax.axis_index("c") * NS + lax.axis_index("s")
        base = wid * QW
        pltpu.sync_copy(zeros_hbm, z_v)
        pltpu.sync_copy(gidx_hbm.at[pl.ds(base, QW)], idx_v)
        pltpu.sync_copy(cexp_hbm.at[pl.ds(base, QW)], c_v)

        def issue(qi, b, sg, sq):
            pltpu.async_copy(T_hbm.at[idx_v.at[qi]], rows_v.at[b], sg)
            pltpu.async_copy(q_hbm.at[pl.ds((base + qi) * H, H)],
                             q_v.at[b], sq)

        def compute(qi, b):
            def per_head(h, _):
                rbase = h * (P * 2)
                iota = lax.iota(jnp.int32, 16)
                cv = c_v[qi, pl.ds(h * 16, 16)]
                cs = [tuple(_lane_gather(cv, jnp.full((16,), p * 4 + t,
                                                      jnp.int32))
                            for t in range(4)) for p in range(P)]
                lvec = jnp.full((16,), -1e30, jnp.float32)
                zv = z_v[...]
                qb = [q_v[b, h, pl.ds(i * 16, 16)] for i in range(NV)]
                for p in range(P):
                    c00, c10, c01, c11 = cs[p]
                    acc = jnp.zeros((16,), jnp.float32)
                    for i in range(NV):
                        k0 = rows_v[b, rbase + 2 * p, pl.ds(Cph + i * 16, 16)]
                        k1 = rows_v[b, rbase + 2 * p + 1,
                                    pl.ds(Cph + i * 16, 16)]
                        ks = (c00 * k0) + (c10 * k1)
                        ks = ks + (c01 * k0)
                        ks = ks + (c11 * k1)
                        acc = acc + qb[i] * _rtne_bf16(ks)
                    tot = _bfly(acc, jnp.add) * _INV_SQRT_CPH
                    lvec = jnp.where(iota == p, tot, lvec)
                m = _bfly(lvec, jnp.maximum)
                e = jnp.exp(lvec - m)
                e = jnp.where(iota < P, e, jnp.zeros((16,), jnp.float32))
                attn = _rtne_bf16(e / _bfly(e, jnp.add))
                avs = [_lane_gather(attn, jnp.full((16,), p, jnp.int32))
                       for p in range(P)]
                for i in range(NV):
                    acc = jnp.zeros((16,), jnp.float32)
                    for p in range(P):
                        c00, c10, c01, c11 = cs[p]
                        v0 = rows_v[b, rbase + 2 * p, pl.ds(i * 16, 16)]
                        v1 = rows_v[b, rbase + 2 * p + 1, pl.ds(i * 16, 16)]
                        vs = (c00 * v0) + (c10 * v1)
                        vs = vs + (c01 * v0)
                        vs = vs + (c11 * v1)
                        acc = acc + avs[p] * _rtne_bf16(vs)
                    out_v[b, h, pl.ds(i * 16, 16)] = acc
                return 0

            lax.fori_loop(0, H, per_head, 0)

        def wait_g(qi, b, sg, sq):
            pltpu.make_async_copy(T_hbm.at[idx_v.at[qi]], rows_v.at[b],
                                  sg).wait()
            pltpu.make_async_copy(q_hbm.at[pl.ds((base + qi) * H, H)],
                                  q_v.at[b], sq).wait()

        def write(qi, b, sw):
            pltpu.async_copy(out_v.at[b],
                             out_hbm.at[pl.ds((base + qi) * H, H)], sw)

        def wait_w(qi, b, sw):
            pltpu.make_async_copy(out_v.at[b],
                                  out_hbm.at[pl.ds((base + qi) * H, H)],
                                  sw).wait()

        issue(0, 0, sg0, sq0)

        def pair(qi2, _):
            q0 = qi2 * 2
            q1 = q0 + 1
            issue(q1, 1, sg1, sq1)
            wait_g(q0, 0, sg0, sq0)

            @pl.when(qi2 >= 1)
            def _():
                wait_w(q0 - 2, 0, sw0)
            compute(q0, 0)
            write(q0, 0, sw0)

            @pl.when(q1 + 1 < QW)
            def _():
                issue(q1 + 1, 0, sg0, sq0)
            wait_g(q1, 1, sg1, sq1)

            @pl.when(qi2 >= 1)
            def _():
                wait_w(q1 - 2, 1, sw1)
            compute(q1, 1)
            write(q1, 1, sw1)
            return 0

        lax.fori_loop(0, QW // 2, pair, 0)
        wait_w(QW - 2, 0, sw0)
        wait_w(QW - 1, 1, sw1)

    return k(T2d, q2d, gidx, cexp, jnp.zeros((16,), jnp.float32))


def kernel(query, key, value, W_ref, b_ref, W_off, b_off, W_v, b_v, W_out, b_out):
    N = query.shape[0]
    M = key.shape[0]

    ref = jax.nn.sigmoid(query @ W_ref.T + b_ref).reshape(N, H, P, 2)
    off = (query @ W_off.T + b_off).reshape(N, H, P, 2)
    loc = ref + off
    x = loc[..., 0] * (M - 1)
    y = loc[..., 1] * (M - 1)
    x0f = jnp.floor(x).astype(jnp.int32)
    y0f = jnp.floor(y).astype(jnp.int32)
    x0 = jnp.clip(x0f, 0, M - 1)
    x1 = jnp.clip(x0f + 1, 0, M - 1)
    y0 = jnp.clip(y0f, 0, M - 1)
    wx = x - x0.astype(jnp.float32)
    wy = y - y0.astype(jnp.float32)
    c00 = (1 - wx) * (1 - wy)
    c10 = wx * (1 - wy)
    c01 = (1 - wx) * wy
    c11 = wx * wy

    h_ar = jnp.arange(H, dtype=jnp.int32)[None, :, None]
    g0 = x0 * H + h_ar
    g1 = x1 * H + h_ar
    gidx = jnp.stack([g0, g1], axis=-1).reshape(N, H * P * 2).astype(jnp.int32)
    cexp = jnp.stack([c00, c10, c01, c11], axis=-1).reshape(N, H * P * 4)

    vproj = _matmul_wt(value, W_v) + b_v
    T = jnp.concatenate(
        [vproj.reshape(M, H, Cph), key.reshape(M, H, Cph)], axis=2
    ).reshape(M * H, TW)

    q_r = query.astype(jnp.bfloat16).astype(jnp.float32)
    out2d = _sc_attend(T, q_r.reshape(N * H, Cph), gidx, cexp, N)

    return _matmul_wt(out2d.reshape(N, C), W_out) + b_out
